# baseline (speedup 1.0000x reference)
import functools

import jax
import jax.numpy as jnp
from jax import lax
from jax.experimental import pallas as pl
from jax.experimental.pallas import tpu as pltpu
from jax.experimental.pallas import tpu_sc as plsc

N = 10000
E = 320000
G = 16
RATIO = 0.8
NEG = -jnp.inf

_EC = 800
_NCHUNK = E // _EC
_NW = 32
_CHUNK_ITERS = (_NCHUNK + _NW - 1) // _NW
_EU = 8
_XPC = 200
_NXP = N // _XPC


def _make_edge_agg(D):
    nv = D // 16
    mesh = plsc.VectorSubcoreMesh(core_axis_name="c", subcore_axis_name="s")

    @functools.partial(
        pl.kernel,
        mesh=mesh,
        out_type=jax.ShapeDtypeStruct((2, N, D), jnp.float32),
        compiler_params=pltpu.CompilerParams(use_tc_tiling_on_sc=False),
        scratch_types=[
            pltpu.VMEM((_EC,), jnp.int32),
            pltpu.VMEM((_EC,), jnp.int32),
            pltpu.VMEM((_EC, 16), jnp.float32),
            pltpu.VMEM((_EC, D), jnp.float32),
            pltpu.VMEM_SHARED((N, D), jnp.float32),
            pltpu.SemaphoreType.DMA,
        ],
    )
    def agg(vals, srci, dsti, eww, out, src_v, dst_v, ew_v, rows_v, acc, sem):
        cid = lax.axis_index("c")
        sid = lax.axis_index("s")
        wid = sid * 2 + cid

        def zrow(r, carry):
            for v in range(nv):
                rows_v[r, pl.ds(v * 16, 16)] = jnp.zeros((16,), jnp.float32)
            return carry
        lax.fori_loop(0, _XPC, zrow, 0)
        for qq in range((_NXP + 15) // 16):
            q = qq * 16 + sid

            @pl.when(q < _NXP)
            def _():
                r0 = pl.multiple_of(q * _XPC, _XPC)
                pltpu.sync_copy(rows_v.at[pl.ds(0, _XPC)], acc.at[pl.ds(r0, _XPC)])
        plsc.subcore_barrier()

        def ebody(eu, carry):
            for u in range(_EU):
                e = eu * _EU + u
                w = ew_v[e, :]
                for v in range(nv):
                    sl = pl.ds(v * 16, 16)
                    rows_v[e, sl] = rows_v[e, sl] * w
            return carry

        def cbody(i, carry):
            j = wid + i * _NW

            @pl.when(j < _NCHUNK)
            def _():
                base = pl.multiple_of(j * _EC, _EC)
                pltpu.sync_copy(srci.at[pl.ds(base, _EC)], src_v)
                pltpu.sync_copy(dsti.at[pl.ds(base, _EC)], dst_v)
                pltpu.sync_copy(eww.at[pl.ds(base, _EC)], ew_v)
                pltpu.async_copy(vals.at[src_v], rows_v, sem).wait()
                lax.fori_loop(0, _EC // _EU, ebody, 0)
                pltpu.sync_copy(rows_v, acc.at[dst_v], add=True)
            return carry

        lax.fori_loop(0, _CHUNK_ITERS, cbody, 0)
        plsc.subcore_barrier()

        for qq in range((_NXP + 15) // 16):
            q = qq * 16 + sid

            @pl.when(q < _NXP)
            def _():
                r0 = pl.multiple_of(q * _XPC, _XPC)
                pltpu.sync_copy(acc.at[pl.ds(r0, _XPC)], out.at[cid, pl.ds(r0, _XPC)])

    return agg


_agg64 = _make_edge_agg(64)
_agg32 = _make_edge_agg(32)


_NP = 10240
_RB = 512


def _rank_body(sc_ref, bc_ref, sr_ref, br_ref, rank_ref, cnt_ref):
    i = pl.program_id(0)
    jc = pl.program_id(1)

    @pl.when(jc == 0)
    def _():
        rank_ref[...] = jnp.zeros_like(rank_ref)
        cnt_ref[...] = jnp.zeros_like(cnt_ref)

    rb = bc_ref[...]
    cb = br_ref[...]
    overlap = (jnp.min(rb) <= jnp.max(cb)) & (jnp.max(rb) >= jnp.min(cb))

    @pl.when(overlap)
    def _():
        rs = sc_ref[...]
        cs = sr_ref[...]
        ri = i * _RB + jax.lax.broadcasted_iota(jnp.int32, (_RB, 1), 0)
        ci = jc * _RB + jax.lax.broadcasted_iota(jnp.int32, (1, _RB), 1)
        meq = (rb == cb)
        mgt = meq & ((cs > rs) | ((cs == rs) & (ci < ri)))
        rank_ref[...] += jnp.sum(mgt.astype(jnp.int32), axis=1, keepdims=True)
        cnt_ref[...] += jnp.sum(meq.astype(jnp.int32), axis=1, keepdims=True)


def _rank_in_graph(score, batch):
    pad = _NP - score.shape[0]
    sp = jnp.concatenate([score, jnp.zeros((pad,), score.dtype)])
    bp = jnp.concatenate([batch, jnp.full((pad,), 0x3FFFFFFF, jnp.int32)])
    grid = (_NP // _RB, _NP // _RB)
    rank, cnt = pl.pallas_call(
        _rank_body,
        grid=grid,
        in_specs=[
            pl.BlockSpec((_RB, 1), lambda i, jc: (i, 0)),
            pl.BlockSpec((_RB, 1), lambda i, jc: (i, 0)),
            pl.BlockSpec((1, _RB), lambda i, jc: (0, jc)),
            pl.BlockSpec((1, _RB), lambda i, jc: (0, jc)),
        ],
        out_specs=[
            pl.BlockSpec((_RB, 1), lambda i, jc: (i, 0)),
            pl.BlockSpec((_RB, 1), lambda i, jc: (i, 0)),
        ],
        out_shape=[
            jax.ShapeDtypeStruct((_NP, 1), jnp.int32),
            jax.ShapeDtypeStruct((_NP, 1), jnp.int32),
        ],
    )(sp.reshape(_NP, 1), bp.reshape(_NP, 1), sp.reshape(1, _NP), bp.reshape(1, _NP))
    n = score.shape[0]
    return rank[:n, 0], cnt[:n, 0]


def kernel(x, edge_index, edge_attr, batch, W_rel1, b_rel1, W_root1, p1,
           W_rel2, b_rel2, W_root2, p2, W_l1, b_l1, W_l2, b_l2, W_l3, b_l3):
    src, dst = edge_index[0], edge_index[1]
    ew = edge_attr

    ewx = jnp.broadcast_to(ew[:, None], (E, 16))
    Pa = _agg64(x[:, :64], src, dst, ewx)
    Pb = _agg64(x[:, 64:], src, dst, ewx)
    agg1 = jnp.concatenate([Pa[0] + Pa[1], Pb[0] + Pb[1]], axis=1)
    h = jax.nn.relu(agg1 @ W_rel1 + b_rel1 + x @ W_root1)
    s1 = (h @ p1) / jnp.linalg.norm(p1)

    rank1, cntn = _rank_in_graph(s1, batch)
    k1n = jnp.ceil(RATIO * cntn.astype(jnp.float32)).astype(jnp.int32)
    k2n = jnp.ceil(RATIO * k1n.astype(jnp.float32)).astype(jnp.int32)
    keep1 = rank1 < k1n
    bounds = jnp.searchsorted(batch, jnp.arange(G + 1, dtype=jnp.int32)).astype(jnp.int32)
    counts = bounds[1:] - bounds[:-1]
    k1 = jnp.ceil(RATIO * counts.astype(jnp.float32)).astype(jnp.int32)
    g1 = jnp.tanh(s1)
    h1 = jnp.where(keep1[:, None], h * g1[:, None], 0.0)
    x1max = jax.ops.segment_max(jnp.where(keep1[:, None], h * g1[:, None], NEG), batch, num_segments=G)
    x1mean = jax.ops.segment_sum(h1, batch, num_segments=G) / jnp.clip(k1.astype(jnp.float32), 1.0)[:, None]
    x1 = jnp.concatenate([x1max, x1mean], axis=1)

    Q = _agg32(h1, src, dst, ewx)
    agg2 = Q[0] + Q[1]
    h2 = jax.nn.relu(agg2 @ W_rel2 + b_rel2 + h1 @ W_root2)
    s2 = (h2 @ p2) / jnp.linalg.norm(p2)

    s2m = jnp.where(keep1, s2, NEG)
    rank2, _ = _rank_in_graph(s2m, batch)
    k2 = jnp.ceil(RATIO * k1.astype(jnp.float32)).astype(jnp.int32)
    keep2 = keep1 & (rank2 < k2n)
    g2 = jnp.tanh(s2)
    h2m = jnp.where(keep2[:, None], h2 * g2[:, None], 0.0)
    x2max = jax.ops.segment_max(jnp.where(keep2[:, None], h2 * g2[:, None], NEG), batch, num_segments=G)
    x2mean = jax.ops.segment_sum(h2m, batch, num_segments=G) / jnp.clip(k2.astype(jnp.float32), 1.0)[:, None]
    x2 = jnp.concatenate([x2max, x2mean], axis=1)

    z = x1 + x2
    z = jax.nn.relu(z @ W_l1 + b_l1)
    z = jax.nn.relu(z @ W_l2 + b_l2)
    z = jax.nn.log_softmax(z @ W_l3 + b_l3, axis=-1)
    return z


# speedup vs baseline: 1.1966x; 1.1966x over previous
import functools

import jax
import jax.numpy as jnp
from jax import lax
from jax.experimental import pallas as pl
from jax.experimental.pallas import tpu as pltpu
from jax.experimental.pallas import tpu_sc as plsc

N = 10000
E = 320000
G = 16
RATIO = 0.8
NEG = -jnp.inf

_EC = 800
_NCHUNK = E // _EC
_NW = 32
_CHUNK_ITERS = (_NCHUNK + _NW - 1) // _NW
_EU = 8
_XPC = 200
_NXP = N // _XPC


def _make_edge_agg(D):
    nv = D // 16
    mesh = plsc.VectorSubcoreMesh(core_axis_name="c", subcore_axis_name="s")

    @functools.partial(
        pl.kernel,
        mesh=mesh,
        out_type=jax.ShapeDtypeStruct((2, N, D), jnp.float32),
        compiler_params=pltpu.CompilerParams(use_tc_tiling_on_sc=False),
        scratch_types=[
            pltpu.VMEM((_EC,), jnp.int32),
            pltpu.VMEM((_EC,), jnp.int32),
            pltpu.VMEM((_EC, 16), jnp.float32),
            pltpu.VMEM((_EC, D), jnp.float32),
            pltpu.VMEM_SHARED((N, D), jnp.float32),
            pltpu.SemaphoreType.DMA,
        ],
    )
    def agg(vals, srci, dsti, eww, out, src_v, dst_v, ew_v, rows_v, acc, sem):
        cid = lax.axis_index("c")
        sid = lax.axis_index("s")
        wid = sid * 2 + cid

        def zrow(r, carry):
            for v in range(nv):
                rows_v[r, pl.ds(v * 16, 16)] = jnp.zeros((16,), jnp.float32)
            return carry
        lax.fori_loop(0, _XPC, zrow, 0)
        for qq in range((_NXP + 15) // 16):
            q = qq * 16 + sid

            @pl.when(q < _NXP)
            def _():
                r0 = pl.multiple_of(q * _XPC, _XPC)
                pltpu.sync_copy(rows_v.at[pl.ds(0, _XPC)], acc.at[pl.ds(r0, _XPC)])
        plsc.subcore_barrier()

        def ebody(eu, carry):
            for u in range(_EU):
                e = eu * _EU + u
                w = ew_v[e, :]
                for v in range(nv):
                    sl = pl.ds(v * 16, 16)
                    rows_v[e, sl] = rows_v[e, sl] * w
            return carry

        def cbody(i, carry):
            j = wid + i * _NW

            @pl.when(j < _NCHUNK)
            def _():
                base = pl.multiple_of(j * _EC, _EC)
                pltpu.sync_copy(srci.at[pl.ds(base, _EC)], src_v)
                pltpu.sync_copy(dsti.at[pl.ds(base, _EC)], dst_v)
                pltpu.sync_copy(eww.at[pl.ds(base, _EC)], ew_v)
                pltpu.async_copy(vals.at[src_v], rows_v, sem).wait()
                lax.fori_loop(0, _EC // _EU, ebody, 0)
                pltpu.sync_copy(rows_v, acc.at[dst_v], add=True)
            return carry

        lax.fori_loop(0, _CHUNK_ITERS, cbody, 0)
        plsc.subcore_barrier()

        for qq in range((_NXP + 15) // 16):
            q = qq * 16 + sid

            @pl.when(q < _NXP)
            def _():
                r0 = pl.multiple_of(q * _XPC, _XPC)
                pltpu.sync_copy(acc.at[pl.ds(r0, _XPC)], out.at[cid, pl.ds(r0, _XPC)])

    return agg


_agg64 = _make_edge_agg(64)
_agg32 = _make_edge_agg(32)


_NP = 10240
_RB = 512
_CB = 2048


def _rank_body(sc_ref, bc_ref, sr_ref, br_ref, rank_ref, cnt_ref):
    i = pl.program_id(0)
    jc = pl.program_id(1)

    @pl.when(jc == 0)
    def _():
        rank_ref[...] = jnp.zeros_like(rank_ref)
        cnt_ref[...] = jnp.zeros_like(cnt_ref)

    rb = bc_ref[...]
    cb = br_ref[...]
    overlap = (jnp.min(rb) <= jnp.max(cb)) & (jnp.max(rb) >= jnp.min(cb))

    @pl.when(overlap)
    def _():
        rs = sc_ref[...]
        cs = sr_ref[...]
        ri = i * _RB + jax.lax.broadcasted_iota(jnp.int32, (_RB, 1), 0)
        ci = jc * _CB + jax.lax.broadcasted_iota(jnp.int32, (1, _CB), 1)
        meq = (rb == cb)
        mgt = meq & ((cs > rs) | ((cs == rs) & (ci < ri)))
        rank_ref[...] += jnp.sum(mgt.astype(jnp.int32), axis=1, keepdims=True)
        cnt_ref[...] += jnp.sum(meq.astype(jnp.int32), axis=1, keepdims=True)


def _rank_in_graph(score, batch):
    pad = _NP - score.shape[0]
    sp = jnp.concatenate([score, jnp.zeros((pad,), score.dtype)])
    bp = jnp.concatenate([batch, jnp.full((pad,), 0x3FFFFFFF, jnp.int32)])
    grid = (_NP // _RB, _NP // _CB)
    rank, cnt = pl.pallas_call(
        _rank_body,
        grid=grid,
        in_specs=[
            pl.BlockSpec((_RB, 1), lambda i, jc: (i, 0)),
            pl.BlockSpec((_RB, 1), lambda i, jc: (i, 0)),
            pl.BlockSpec((1, _CB), lambda i, jc: (0, jc)),
            pl.BlockSpec((1, _CB), lambda i, jc: (0, jc)),
        ],
        out_specs=[
            pl.BlockSpec((_RB, 1), lambda i, jc: (i, 0)),
            pl.BlockSpec((_RB, 1), lambda i, jc: (i, 0)),
        ],
        out_shape=[
            jax.ShapeDtypeStruct((_NP, 1), jnp.int32),
            jax.ShapeDtypeStruct((_NP, 1), jnp.int32),
        ],
    )(sp.reshape(_NP, 1), bp.reshape(_NP, 1), sp.reshape(1, _NP), bp.reshape(1, _NP))
    n = score.shape[0]
    return rank[:n, 0], cnt[:n, 0]


def kernel(x, edge_index, edge_attr, batch, W_rel1, b_rel1, W_root1, p1,
           W_rel2, b_rel2, W_root2, p2, W_l1, b_l1, W_l2, b_l2, W_l3, b_l3):
    src, dst = edge_index[0], edge_index[1]
    ew = edge_attr

    ewx = jnp.broadcast_to(ew[:, None], (E, 16))
    Pa = _agg64(x[:, :64], src, dst, ewx)
    Pb = _agg64(x[:, 64:], src, dst, ewx)
    agg1 = jnp.concatenate([Pa[0] + Pa[1], Pb[0] + Pb[1]], axis=1)
    h = jax.nn.relu(agg1 @ W_rel1 + b_rel1 + x @ W_root1)
    s1 = (h @ p1) / jnp.linalg.norm(p1)

    rank1, cntn = _rank_in_graph(s1, batch)
    k1n = jnp.ceil(RATIO * cntn.astype(jnp.float32)).astype(jnp.int32)
    k2n = jnp.ceil(RATIO * k1n.astype(jnp.float32)).astype(jnp.int32)
    keep1 = rank1 < k1n
    bounds = jnp.searchsorted(batch, jnp.arange(G + 1, dtype=jnp.int32)).astype(jnp.int32)
    counts = bounds[1:] - bounds[:-1]
    k1 = jnp.ceil(RATIO * counts.astype(jnp.float32)).astype(jnp.int32)
    g1 = jnp.tanh(s1)
    h1 = jnp.where(keep1[:, None], h * g1[:, None], 0.0)
    x1max = jax.ops.segment_max(jnp.where(keep1[:, None], h * g1[:, None], NEG), batch, num_segments=G)
    x1mean = jax.ops.segment_sum(h1, batch, num_segments=G) / jnp.clip(k1.astype(jnp.float32), 1.0)[:, None]
    x1 = jnp.concatenate([x1max, x1mean], axis=1)

    Q = _agg32(h1, src, dst, ewx)
    agg2 = Q[0] + Q[1]
    h2 = jax.nn.relu(agg2 @ W_rel2 + b_rel2 + h1 @ W_root2)
    s2 = (h2 @ p2) / jnp.linalg.norm(p2)

    s2m = jnp.where(keep1, s2, NEG)
    rank2, _ = _rank_in_graph(s2m, batch)
    k2 = jnp.ceil(RATIO * k1.astype(jnp.float32)).astype(jnp.int32)
    keep2 = keep1 & (rank2 < k2n)
    g2 = jnp.tanh(s2)
    h2m = jnp.where(keep2[:, None], h2 * g2[:, None], 0.0)
    x2max = jax.ops.segment_max(jnp.where(keep2[:, None], h2 * g2[:, None], NEG), batch, num_segments=G)
    x2mean = jax.ops.segment_sum(h2m, batch, num_segments=G) / jnp.clip(k2.astype(jnp.float32), 1.0)[:, None]
    x2 = jnp.concatenate([x2max, x2mean], axis=1)

    z = x1 + x2
    z = jax.nn.relu(z @ W_l1 + b_l1)
    z = jax.nn.relu(z @ W_l2 + b_l2)
    z = jax.nn.log_softmax(z @ W_l3 + b_l3, axis=-1)
    return z


# speedup vs baseline: 1.3610x; 1.1374x over previous
import functools

import jax
import jax.numpy as jnp
from jax import lax
from jax.experimental import pallas as pl
from jax.experimental.pallas import tpu as pltpu
from jax.experimental.pallas import tpu_sc as plsc

N = 10000
E = 320000
G = 16
RATIO = 0.8
NEG = -jnp.inf

_EC = 800
_NCHUNK = E // _EC
_NW = 32
_CHUNK_ITERS = (_NCHUNK + _NW - 1) // _NW
_EU = 8
_XPC = 200
_NXP = N // _XPC


def _make_edge_agg(D):
    nv = D // 16
    mesh = plsc.VectorSubcoreMesh(core_axis_name="c", subcore_axis_name="s")

    @functools.partial(
        pl.kernel,
        mesh=mesh,
        out_type=jax.ShapeDtypeStruct((2, N, D), jnp.float32),
        compiler_params=pltpu.CompilerParams(use_tc_tiling_on_sc=False),
        scratch_types=[
            pltpu.VMEM((_EC,), jnp.int32),
            pltpu.VMEM((_EC,), jnp.int32),
            pltpu.VMEM((_EC, 16), jnp.float32),
            pltpu.VMEM((_EC, D), jnp.float32),
            pltpu.VMEM_SHARED((N, D), jnp.float32),
            pltpu.SemaphoreType.DMA,
        ],
    )
    def agg(vals, srci, dsti, eww, out, src_v, dst_v, ew_v, rows_v, acc, sem):
        cid = lax.axis_index("c")
        sid = lax.axis_index("s")
        wid = sid * 2 + cid

        def zrow(r, carry):
            for v in range(nv):
                rows_v[r, pl.ds(v * 16, 16)] = jnp.zeros((16,), jnp.float32)
            return carry
        lax.fori_loop(0, _XPC, zrow, 0)
        for qq in range((_NXP + 15) // 16):
            q = qq * 16 + sid

            @pl.when(q < _NXP)
            def _():
                r0 = pl.multiple_of(q * _XPC, _XPC)
                pltpu.sync_copy(rows_v.at[pl.ds(0, _XPC)], acc.at[pl.ds(r0, _XPC)])
        plsc.subcore_barrier()

        def ebody(eu, carry):
            for u in range(_EU):
                e = eu * _EU + u
                w = ew_v[e, :]
                for v in range(nv):
                    sl = pl.ds(v * 16, 16)
                    rows_v[e, sl] = rows_v[e, sl] * w
            return carry

        def cbody(i, carry):
            j = wid + i * _NW

            @pl.when(j < _NCHUNK)
            def _():
                base = pl.multiple_of(j * _EC, _EC)
                pltpu.sync_copy(srci.at[pl.ds(base, _EC)], src_v)
                pltpu.sync_copy(dsti.at[pl.ds(base, _EC)], dst_v)
                pltpu.sync_copy(eww.at[pl.ds(base, _EC)], ew_v)
                pltpu.async_copy(vals.at[src_v], rows_v, sem).wait()
                lax.fori_loop(0, _EC // _EU, ebody, 0)
                pltpu.sync_copy(rows_v, acc.at[dst_v], add=True)
            return carry

        lax.fori_loop(0, _CHUNK_ITERS, cbody, 0)
        plsc.subcore_barrier()

        for qq in range((_NXP + 15) // 16):
            q = qq * 16 + sid

            @pl.when(q < _NXP)
            def _():
                r0 = pl.multiple_of(q * _XPC, _XPC)
                pltpu.sync_copy(acc.at[pl.ds(r0, _XPC)], out.at[cid, pl.ds(r0, _XPC)])

    return agg


_agg64 = _make_edge_agg(64)
_agg32 = _make_edge_agg(32)


_NP = 10240
_RB = 512
_CB = 2048


def _rank_body(sc_ref, bc_ref, sr_ref, br_ref, rank_ref, cnt_ref):
    i = pl.program_id(0)
    jc = pl.program_id(1)

    @pl.when(jc == 0)
    def _():
        rank_ref[...] = jnp.zeros_like(rank_ref)
        cnt_ref[...] = jnp.zeros_like(cnt_ref)

    rb = bc_ref[...]
    cb = br_ref[...]
    overlap = (jnp.min(rb) <= jnp.max(cb)) & (jnp.max(rb) >= jnp.min(cb))

    @pl.when(overlap)
    def _():
        rs = sc_ref[...]
        cs = sr_ref[...]
        ri = i * _RB + jax.lax.broadcasted_iota(jnp.int32, (_RB, 1), 0)
        ci = jc * _CB + jax.lax.broadcasted_iota(jnp.int32, (1, _CB), 1)
        meq = (rb == cb)
        mgt = meq & ((cs > rs) | ((cs == rs) & (ci < ri)))
        rank_ref[...] += jnp.sum(mgt.astype(jnp.int32), axis=1, keepdims=True)
        cnt_ref[...] += jnp.sum(meq.astype(jnp.int32), axis=1, keepdims=True)


def _rank_in_graph(score, batch):
    pad = _NP - score.shape[0]
    sp = jnp.concatenate([score, jnp.zeros((pad,), score.dtype)])
    bp = jnp.concatenate([batch, jnp.full((pad,), 0x3FFFFFFF, jnp.int32)])
    grid = (_NP // _RB, _NP // _CB)
    rank, cnt = pl.pallas_call(
        _rank_body,
        grid=grid,
        in_specs=[
            pl.BlockSpec((_RB, 1), lambda i, jc: (i, 0)),
            pl.BlockSpec((_RB, 1), lambda i, jc: (i, 0)),
            pl.BlockSpec((1, _CB), lambda i, jc: (0, jc)),
            pl.BlockSpec((1, _CB), lambda i, jc: (0, jc)),
        ],
        out_specs=[
            pl.BlockSpec((_RB, 1), lambda i, jc: (i, 0)),
            pl.BlockSpec((_RB, 1), lambda i, jc: (i, 0)),
        ],
        out_shape=[
            jax.ShapeDtypeStruct((_NP, 1), jnp.int32),
            jax.ShapeDtypeStruct((_NP, 1), jnp.int32),
        ],
    )(sp.reshape(_NP, 1), bp.reshape(_NP, 1), sp.reshape(1, _NP), bp.reshape(1, _NP))
    n = score.shape[0]
    return rank[:n, 0], cnt[:n, 0]


def _pool_body(hg_ref, b_ref, k_ref, mx_ref, sm_ref, kc_ref):
    i = pl.program_id(0)

    @pl.when(i == 0)
    def _():
        mx_ref[...] = jnp.full_like(mx_ref, NEG)
        sm_ref[...] = jnp.zeros_like(sm_ref)
        kc_ref[...] = jnp.zeros_like(kc_ref)

    hg = hg_ref[...]
    b = b_ref[...]
    kept = k_ref[...] != 0
    for g in range(G):
        mg = (b == g) & kept
        mx_ref[g:g + 1, :] = jnp.maximum(
            mx_ref[g:g + 1, :],
            jnp.max(jnp.where(mg, hg, NEG), axis=0, keepdims=True))
        sm_ref[g:g + 1, :] += jnp.sum(jnp.where(mg, hg, 0.0), axis=0, keepdims=True)
        kc_ref[g:g + 1, :] += jnp.sum(mg.astype(jnp.float32), axis=0, keepdims=True)


def _pool(hg, batch, keep):
    pad = _NP - hg.shape[0]
    hgp = jnp.concatenate([hg, jnp.zeros((pad, hg.shape[1]), hg.dtype)])
    bp = jnp.concatenate([batch, jnp.full((pad,), 0x3FFFFFFF, jnp.int32)])
    kp = jnp.concatenate([keep.astype(jnp.int32), jnp.zeros((pad,), jnp.int32)])
    mx, sm, kc = pl.pallas_call(
        _pool_body,
        grid=(_NP // _RB,),
        in_specs=[
            pl.BlockSpec((_RB, 32), lambda i: (i, 0)),
            pl.BlockSpec((_RB, 1), lambda i: (i, 0)),
            pl.BlockSpec((_RB, 1), lambda i: (i, 0)),
        ],
        out_specs=[
            pl.BlockSpec((G, 32), lambda i: (0, 0)),
            pl.BlockSpec((G, 32), lambda i: (0, 0)),
            pl.BlockSpec((G, 1), lambda i: (0, 0)),
        ],
        out_shape=[
            jax.ShapeDtypeStruct((G, 32), jnp.float32),
            jax.ShapeDtypeStruct((G, 32), jnp.float32),
            jax.ShapeDtypeStruct((G, 1), jnp.float32),
        ],
    )(hgp, bp.reshape(_NP, 1), kp.reshape(_NP, 1))
    return mx, sm, kc


def kernel(x, edge_index, edge_attr, batch, W_rel1, b_rel1, W_root1, p1,
           W_rel2, b_rel2, W_root2, p2, W_l1, b_l1, W_l2, b_l2, W_l3, b_l3):
    src, dst = edge_index[0], edge_index[1]
    ew = edge_attr

    ewx = jnp.broadcast_to(ew[:, None], (E, 16))
    Pa = _agg64(x[:, :64], src, dst, ewx)
    Pb = _agg64(x[:, 64:], src, dst, ewx)
    agg1 = jnp.concatenate([Pa[0] + Pa[1], Pb[0] + Pb[1]], axis=1)
    h = jax.nn.relu(agg1 @ W_rel1 + b_rel1 + x @ W_root1)
    s1 = (h @ p1) / jnp.linalg.norm(p1)

    rank1, cntn = _rank_in_graph(s1, batch)
    k1n = jnp.ceil(RATIO * cntn.astype(jnp.float32)).astype(jnp.int32)
    k2n = jnp.ceil(RATIO * k1n.astype(jnp.float32)).astype(jnp.int32)
    keep1 = rank1 < k1n
    g1 = jnp.tanh(s1)
    hg1 = h * g1[:, None]
    h1 = jnp.where(keep1[:, None], hg1, 0.0)
    x1max, x1sum, kc1 = _pool(hg1, batch, keep1)
    x1mean = x1sum / jnp.clip(kc1, 1.0)
    x1 = jnp.concatenate([x1max, x1mean], axis=1)

    Q = _agg32(h1, src, dst, ewx)
    agg2 = Q[0] + Q[1]
    h2 = jax.nn.relu(agg2 @ W_rel2 + b_rel2 + h1 @ W_root2)
    s2 = (h2 @ p2) / jnp.linalg.norm(p2)

    s2m = jnp.where(keep1, s2, NEG)
    rank2, _ = _rank_in_graph(s2m, batch)
    keep2 = keep1 & (rank2 < k2n)
    g2 = jnp.tanh(s2)
    hg2 = h2 * g2[:, None]
    x2max, x2sum, kc2 = _pool(hg2, batch, keep2)
    x2mean = x2sum / jnp.clip(kc2, 1.0)
    x2 = jnp.concatenate([x2max, x2mean], axis=1)

    z = x1 + x2
    z = jax.nn.relu(z @ W_l1 + b_l1)
    z = jax.nn.relu(z @ W_l2 + b_l2)
    z = jax.nn.log_softmax(z @ W_l3 + b_l3, axis=-1)
    return z
